# baseline (speedup 1.0000x reference)
import jax
import jax.numpy as jnp
from jax import lax
from jax.experimental import pallas as pl
from jax.experimental.pallas import tpu as pltpu
from jax.experimental.pallas import tpu_sc as plsc

_LANES = 128
_CHUNK = 128


def _make_fused(n_rows: int, d: int):
  info = plsc.get_sparse_core_info()
  nc, ns = info.num_cores, info.num_subcores
  nw = nc * ns
  rows_w = n_rows // nw
  nchunk = rows_w // _CHUNK
  gch = rows_w // _LANES
  nvec = d // 16

  mesh = plsc.VectorSubcoreMesh(core_axis_name="c", subcore_axis_name="s")

  def body(x_hbm, w_hbm, b_hbm, idx_hbm, out_hbm,
           idx_v, w_v, b_v, xb0, xb1, xb2, ob0, ob1,
           gsem, xsem0, xsem1, xsem2, osem0, osem1):
    wid = lax.axis_index("s") * nc + lax.axis_index("c")
    row0 = wid * rows_w

    pltpu.sync_copy(idx_hbm.at[pl.ds(wid * gch, gch)], idx_v)
    gathers = []
    for j in range(gch):
      gathers.append(pltpu.async_copy(
          w_hbm.at[idx_v.at[j]], w_v.at[pl.ds(j * _LANES, _LANES)], gsem))
      gathers.append(pltpu.async_copy(
          b_hbm.at[idx_v.at[j]], b_v.at[pl.ds(j * _LANES, _LANES)], gsem))

    xbufs, obufs = [xb0, xb1, xb2], [ob0, ob1]
    xsems, osems = [xsem0, xsem1, xsem2], [osem0, osem1]

    xcopies = [None] * nchunk
    for p in range(min(2, nchunk)):
      xcopies[p] = pltpu.async_copy(
          x_hbm.at[pl.ds(row0 + p * _CHUNK, _CHUNK), :], xbufs[p % 3],
          xsems[p % 3])
    for g in gathers:
      g.wait()

    ocopies = [None] * nchunk
    for k in range(nchunk):
      if k + 2 < nchunk:
        xcopies[k + 2] = pltpu.async_copy(
            x_hbm.at[pl.ds(row0 + (k + 2) * _CHUNK, _CHUNK), :],
            xbufs[(k + 2) % 3], xsems[(k + 2) % 3])
      xcopies[k].wait()
      if k >= 2:
        ocopies[k - 2].wait()
      xb, ob = xbufs[k % 3], obufs[k % 2]

      def grp_body(g, carry, k=k, xb=xb, ob=ob):
        base = g * 16
        w16 = w_v[pl.ds(k * _CHUNK + base, 16)]
        b16 = b_v[pl.ds(k * _CHUNK + base, 16)]
        for l in range(16):
          li = jnp.full((16,), l, jnp.int32)
          sv = w16.at[li].get(mode="promise_in_bounds")
          tv = b16.at[li].get(mode="promise_in_bounds")
          r = base + l
          for v in range(nvec):
            ob[r, pl.ds(v * 16, 16)] = xb[r, pl.ds(v * 16, 16)] * sv + tv
        return carry

      lax.fori_loop(0, _CHUNK // 16, grp_body, 0)
      ocopies[k] = pltpu.async_copy(
          ob, out_hbm.at[pl.ds(row0 + k * _CHUNK, _CHUNK), :], osems[k % 2])

    for k in range(max(0, nchunk - 2), nchunk):
      ocopies[k].wait()

  return pl.kernel(
      body,
      out_type=jax.ShapeDtypeStruct((n_rows, d), jnp.float32),
      mesh=mesh,
      scratch_types=[
          pltpu.VMEM((gch, _LANES), jnp.int32),
          pltpu.VMEM((rows_w,), jnp.float32),
          pltpu.VMEM((rows_w,), jnp.float32),
          pltpu.VMEM((_CHUNK, d), jnp.float32),
          pltpu.VMEM((_CHUNK, d), jnp.float32),
          pltpu.VMEM((_CHUNK, d), jnp.float32),
          pltpu.VMEM((_CHUNK, d), jnp.float32),
          pltpu.VMEM((_CHUNK, d), jnp.float32),
          pltpu.SemaphoreType.DMA,
          pltpu.SemaphoreType.DMA,
          pltpu.SemaphoreType.DMA,
          pltpu.SemaphoreType.DMA,
          pltpu.SemaphoreType.DMA,
          pltpu.SemaphoreType.DMA,
      ],
  )


def kernel(x, batch_index, weight, bias):
  n_rows, d = x.shape
  idx = batch_index.reshape(n_rows // _LANES, _LANES).astype(jnp.int32)
  return _make_fused(n_rows, d)(x, weight, bias, idx)
